# baseline (speedup 1.0000x reference)
import functools
import jax
import jax.numpy as jnp
from jax import lax
from jax.experimental import pallas as pl
from jax.experimental.pallas import tpu as pltpu
from jax.experimental.pallas import tpu_sc as plsc

N = 10000
E = 320000
D = 128
NC = 2
NS = 16
NW = NC * NS
EPW = E // NW
K = 50
NP_ = 2
CH = EPW // (K * NP_)
NPAD = 10240

_mesh = plsc.VectorSubcoreMesh(core_axis_name="c", subcore_axis_name="s")


@functools.partial(
    pl.kernel,
    out_type=jax.ShapeDtypeStruct((NC * NPAD,), jnp.float32),
    mesh=_mesh,
    scratch_types=[
        pltpu.VMEM((CH, K), jnp.int32),
        pltpu.VMEM((128,), jnp.float32),
        pltpu.VMEM((640,), jnp.float32),
        pltpu.VMEM_SHARED((NPAD,), jnp.float32),
        pltpu.SemaphoreType.DMA,
    ],
)
def _hist(dst3_hbm, out_hbm, dsts_v, ones_v, zer_v, hist_sh, sem):
    c = lax.axis_index("c")
    s = lax.axis_index("s")
    wid = s * NC + c
    for i in range(8):
        ones_v[pl.ds(i * 16, 16)] = jnp.ones((16,), jnp.float32)
    for i in range(640 // 16):
        zer_v[pl.ds(i * 16, 16)] = jnp.zeros((16,), jnp.float32)
    pltpu.sync_copy(zer_v, hist_sh.at[pl.ds(s * 640, 640)])
    plsc.subcore_barrier()
    ones_k = ones_v.at[pl.ds(0, K)]

    def grp(g, carry):
        for b in range(5):
            pltpu.async_copy(ones_k, hist_sh.at[dsts_v.at[5 * g + b]], sem, add=True)
        for b in range(5):
            pltpu.make_async_copy(ones_k, hist_sh.at[dsts_v.at[0]], sem).wait()
        return carry

    for p in range(NP_):
        pltpu.sync_copy(dst3_hbm.at[NP_ * wid + p], dsts_v)
        lax.fori_loop(0, CH // 5, grp, 0)
    plsc.subcore_barrier()
    pltpu.sync_copy(hist_sh.at[pl.ds(s * 640, 640)],
                    out_hbm.at[pl.ds(c * NPAD + s * 640, 640)])


@functools.partial(
    pl.kernel,
    out_type=jax.ShapeDtypeStruct((NC, NPAD, D), jnp.float32),
    mesh=_mesh,
    scratch_types=[
        pltpu.VMEM((CH, K), jnp.int32),
        pltpu.VMEM((CH, K), jnp.int32),
        pltpu.VMEM((K, D), jnp.float32),
        pltpu.VMEM((K, D), jnp.float32),
        pltpu.VMEM((K, D), jnp.float32),
        pltpu.VMEM_SHARED((NPAD, D), jnp.float32),
        pltpu.SemaphoreType.DMA,
        pltpu.SemaphoreType.DMA,
        pltpu.SemaphoreType.DMA,
        pltpu.SemaphoreType.DMA,
    ],
)
def _agg(y_hbm, src3_hbm, dst3_hbm, out_hbm, srcs_v, dsts_v, rows_a, rows_b,
         rows_c, acc_sh, sga, sgb, sgc, sout):
    c = lax.axis_index("c")
    s = lax.axis_index("s")
    wid = s * NC + c

    def zf(r, carry):
        for j in range(D // 16):
            rows_a[r, pl.ds(j * 16, 16)] = jnp.zeros((16,), jnp.float32)
        return carry

    lax.fori_loop(0, 80, zf, 0)
    zblk = rows_a.at[pl.ds(0, 80)]
    for k in range(8):
        pltpu.sync_copy(zblk, acc_sh.at[pl.ds(s * 640 + k * 80, 80)])
    plsc.subcore_barrier()

    def step(g, carry):
        i = 3 * g
        pltpu.make_async_copy(y_hbm.at[srcs_v.at[0]], rows_a, sga).wait()
        pltpu.sync_copy(rows_a, acc_sh.at[dsts_v.at[i]], add=True)
        pltpu.async_copy(y_hbm.at[srcs_v.at[i + 3]], rows_a, sga)
        pltpu.make_async_copy(y_hbm.at[srcs_v.at[0]], rows_b, sgb).wait()
        pltpu.sync_copy(rows_b, acc_sh.at[dsts_v.at[i + 1]], add=True)
        pltpu.async_copy(y_hbm.at[srcs_v.at[i + 4]], rows_b, sgb)
        pltpu.make_async_copy(y_hbm.at[srcs_v.at[0]], rows_c, sgc).wait()
        pltpu.sync_copy(rows_c, acc_sh.at[dsts_v.at[i + 2]], add=True)
        pltpu.async_copy(y_hbm.at[srcs_v.at[i + 5]], rows_c, sgc)
        return carry

    for p in range(NP_):
        pltpu.sync_copy(src3_hbm.at[NP_ * wid + p], srcs_v)
        pltpu.sync_copy(dst3_hbm.at[NP_ * wid + p], dsts_v)
        pltpu.async_copy(y_hbm.at[srcs_v.at[0]], rows_a, sga)
        pltpu.async_copy(y_hbm.at[srcs_v.at[1]], rows_b, sgb)
        pltpu.async_copy(y_hbm.at[srcs_v.at[2]], rows_c, sgc)
        lax.fori_loop(0, CH // 3 - 1, step, 0)
        pltpu.make_async_copy(y_hbm.at[srcs_v.at[0]], rows_a, sga).wait()
        pltpu.sync_copy(rows_a, acc_sh.at[dsts_v.at[CH - 4]], add=True)
        pltpu.async_copy(y_hbm.at[srcs_v.at[CH - 1]], rows_a, sga)
        pltpu.make_async_copy(y_hbm.at[srcs_v.at[0]], rows_b, sgb).wait()
        pltpu.sync_copy(rows_b, acc_sh.at[dsts_v.at[CH - 3]], add=True)
        pltpu.make_async_copy(y_hbm.at[srcs_v.at[0]], rows_c, sgc).wait()
        pltpu.sync_copy(rows_c, acc_sh.at[dsts_v.at[CH - 2]], add=True)
        pltpu.make_async_copy(y_hbm.at[srcs_v.at[0]], rows_a, sga).wait()
        pltpu.sync_copy(rows_a, acc_sh.at[dsts_v.at[CH - 1]], add=True)
    plsc.subcore_barrier()
    for k in range(5):
        off = s * 640 + k * 128
        pltpu.async_copy(acc_sh.at[pl.ds(off, 128)], out_hbm.at[c, pl.ds(off, 128)], sout)
    for k in range(5):
        off = s * 640 + k * 128
        pltpu.make_async_copy(acc_sh.at[pl.ds(off, 128)], out_hbm.at[c, pl.ds(off, 128)], sout).wait()


R = 2000
G = N // R


def _mm_body(x_ref, w_ref, o_ref):
    o_ref[...] = jnp.dot(x_ref[...], w_ref[...], preferred_element_type=jnp.float32)


def _tc1_body(hist_ref, xw_ref, y_ref):
    dinv = lax.rsqrt(hist_ref[0] + hist_ref[1] + 1.0)
    y_ref[...] = dinv * xw_ref[...]


def _tc2_body(hist_ref, agg_ref, y1_ref, b1_ref, w2_ref, y2_ref):
    dinv = lax.rsqrt(hist_ref[0] + hist_ref[1] + 1.0)
    pre = agg_ref[0] + agg_ref[1] + y1_ref[...]
    h = jnp.maximum(dinv * pre + b1_ref[...], 0.0)
    y2_ref[...] = dinv * jnp.dot(h, w2_ref[...], preferred_element_type=jnp.float32)


def _tc3_body(hist_ref, agg_ref, y2_ref, b2_ref, out_ref):
    dinv = lax.rsqrt(hist_ref[0] + hist_ref[1] + 1.0)
    out_ref[...] = dinv * (agg_ref[0] + agg_ref[1] + y2_ref[...]) + b2_ref[...]


_hist_spec = pl.BlockSpec((2, R, 1), lambda i: (0, i, 0))
_row_spec = pl.BlockSpec((R, D), lambda i: (i, 0))
_agg_spec = pl.BlockSpec((2, R, D), lambda i: (0, i, 0))
_w_spec = pl.BlockSpec((D, D), lambda i: (0, 0))
_b_spec = pl.BlockSpec((1, D), lambda i: (0, 0))
_row_out = jax.ShapeDtypeStruct((N, D), jnp.float32)

_mm = pl.pallas_call(
    _mm_body, grid=(G,),
    in_specs=[_row_spec, _w_spec],
    out_specs=_row_spec, out_shape=_row_out,
)
_tc1 = pl.pallas_call(
    _tc1_body, grid=(G,),
    in_specs=[_hist_spec, _row_spec],
    out_specs=_row_spec, out_shape=_row_out,
)
_tc2 = pl.pallas_call(
    _tc2_body, grid=(G,),
    in_specs=[_hist_spec, _agg_spec, _row_spec, _b_spec, _w_spec],
    out_specs=_row_spec, out_shape=_row_out,
)
_tc3 = pl.pallas_call(
    _tc3_body, grid=(G,),
    in_specs=[_hist_spec, _agg_spec, _row_spec, _b_spec],
    out_specs=_row_spec, out_shape=_row_out,
)


@jax.jit
def _impl(x, edge_index, W1, b1, W2, b2):
    src3 = edge_index[0].reshape(NW * NP_, CH, K)
    dst3 = edge_index[1].reshape(NW * NP_, CH, K)
    xw1 = _mm(x, W1)
    histp = _hist(dst3)
    hist = histp.reshape(2, NPAD, 1)[:, :N]
    y1 = _tc1(hist, xw1)
    agg1 = _agg(y1, src3, dst3)
    y2 = _tc2(hist, agg1, y1, b1.reshape(1, D), W2)
    agg2 = _agg(y2, src3, dst3)
    return _tc3(hist, agg2, y2, b2.reshape(1, D))


def kernel(x, edge_index, W1, b1, W2, b2):
    return _impl(x, edge_index, W1, b1, W2, b2)


# speedup vs baseline: 1.0028x; 1.0028x over previous
import functools
import jax
import jax.numpy as jnp
from jax import lax
from jax.experimental import pallas as pl
from jax.experimental.pallas import tpu as pltpu
from jax.experimental.pallas import tpu_sc as plsc

N = 10000
E = 320000
D = 128
NC = 2
NS = 16
NW = NC * NS
EPW = E // NW
K = 50
NP_ = 2
CH = EPW // (K * NP_)
NPAD = 10240

_mesh = plsc.VectorSubcoreMesh(core_axis_name="c", subcore_axis_name="s")


@functools.partial(
    pl.kernel,
    out_type=jax.ShapeDtypeStruct((NC * NPAD,), jnp.float32),
    mesh=_mesh,
    scratch_types=[
        pltpu.VMEM((CH, K), jnp.int32),
        pltpu.VMEM((128,), jnp.float32),
        pltpu.VMEM((640,), jnp.float32),
        pltpu.VMEM_SHARED((NPAD,), jnp.float32),
        pltpu.SemaphoreType.DMA,
    ],
)
def _hist(dst3_hbm, out_hbm, dsts_v, ones_v, zer_v, hist_sh, sem):
    c = lax.axis_index("c")
    s = lax.axis_index("s")
    wid = s * NC + c
    for i in range(8):
        ones_v[pl.ds(i * 16, 16)] = jnp.ones((16,), jnp.float32)
    for i in range(640 // 16):
        zer_v[pl.ds(i * 16, 16)] = jnp.zeros((16,), jnp.float32)
    pltpu.sync_copy(zer_v, hist_sh.at[pl.ds(s * 640, 640)])
    plsc.subcore_barrier()
    ones_k = ones_v.at[pl.ds(0, K)]

    def grp(g, carry):
        for b in range(5):
            pltpu.async_copy(ones_k, hist_sh.at[dsts_v.at[5 * g + b]], sem, add=True)
        for b in range(5):
            pltpu.make_async_copy(ones_k, hist_sh.at[dsts_v.at[0]], sem).wait()
        return carry

    for p in range(NP_):
        pltpu.sync_copy(dst3_hbm.at[NP_ * wid + p], dsts_v)
        lax.fori_loop(0, CH // 5, grp, 0)
    plsc.subcore_barrier()
    pltpu.sync_copy(hist_sh.at[pl.ds(s * 640, 640)],
                    out_hbm.at[pl.ds(c * NPAD + s * 640, 640)])


@functools.partial(
    pl.kernel,
    out_type=jax.ShapeDtypeStruct((NC, NPAD, D), jnp.float32),
    mesh=_mesh,
    scratch_types=[
        pltpu.VMEM((CH, K), jnp.int32),
        pltpu.VMEM((CH, K), jnp.int32),
        pltpu.VMEM((K, D), jnp.float32),
        pltpu.VMEM((K, D), jnp.float32),
        pltpu.VMEM((K, D), jnp.float32),
        pltpu.VMEM_SHARED((NPAD, D), jnp.float32),
        pltpu.SemaphoreType.DMA,
        pltpu.SemaphoreType.DMA,
        pltpu.SemaphoreType.DMA,
        pltpu.SemaphoreType.DMA,
    ],
)
def _agg(y_hbm, src3_hbm, dst3_hbm, out_hbm, srcs_v, dsts_v, rows_a, rows_b,
         rows_c, acc_sh, sga, sgb, sgc, sout):
    c = lax.axis_index("c")
    s = lax.axis_index("s")
    wid = s * NC + c

    def zf(r, carry):
        for j in range(D // 16):
            rows_a[r, pl.ds(j * 16, 16)] = jnp.zeros((16,), jnp.float32)
        return carry

    lax.fori_loop(0, 80, zf, 0)
    zblk = rows_a.at[pl.ds(0, 80)]
    for k in range(8):
        pltpu.sync_copy(zblk, acc_sh.at[pl.ds(s * 640 + k * 80, 80)])
    plsc.subcore_barrier()

    def step(g, carry):
        i = 3 * g
        pltpu.make_async_copy(y_hbm.at[srcs_v.at[0]], rows_a, sga).wait()
        pltpu.sync_copy(rows_a, acc_sh.at[dsts_v.at[i]], add=True)
        pltpu.async_copy(y_hbm.at[srcs_v.at[i + 3]], rows_a, sga)
        pltpu.make_async_copy(y_hbm.at[srcs_v.at[0]], rows_b, sgb).wait()
        pltpu.sync_copy(rows_b, acc_sh.at[dsts_v.at[i + 1]], add=True)
        pltpu.async_copy(y_hbm.at[srcs_v.at[i + 4]], rows_b, sgb)
        pltpu.make_async_copy(y_hbm.at[srcs_v.at[0]], rows_c, sgc).wait()
        pltpu.sync_copy(rows_c, acc_sh.at[dsts_v.at[i + 2]], add=True)
        pltpu.async_copy(y_hbm.at[srcs_v.at[i + 5]], rows_c, sgc)
        return carry

    for p in range(NP_):
        pltpu.sync_copy(src3_hbm.at[NP_ * wid + p], srcs_v)
        pltpu.sync_copy(dst3_hbm.at[NP_ * wid + p], dsts_v)
        pltpu.async_copy(y_hbm.at[srcs_v.at[0]], rows_a, sga)
        pltpu.async_copy(y_hbm.at[srcs_v.at[1]], rows_b, sgb)
        pltpu.async_copy(y_hbm.at[srcs_v.at[2]], rows_c, sgc)
        lax.fori_loop(0, CH // 3 - 1, step, 0)
        pltpu.make_async_copy(y_hbm.at[srcs_v.at[0]], rows_a, sga).wait()
        pltpu.sync_copy(rows_a, acc_sh.at[dsts_v.at[CH - 4]], add=True)
        pltpu.async_copy(y_hbm.at[srcs_v.at[CH - 1]], rows_a, sga)
        pltpu.make_async_copy(y_hbm.at[srcs_v.at[0]], rows_b, sgb).wait()
        pltpu.sync_copy(rows_b, acc_sh.at[dsts_v.at[CH - 3]], add=True)
        pltpu.make_async_copy(y_hbm.at[srcs_v.at[0]], rows_c, sgc).wait()
        pltpu.sync_copy(rows_c, acc_sh.at[dsts_v.at[CH - 2]], add=True)
        pltpu.make_async_copy(y_hbm.at[srcs_v.at[0]], rows_a, sga).wait()
        pltpu.sync_copy(rows_a, acc_sh.at[dsts_v.at[CH - 1]], add=True)
    plsc.subcore_barrier()
    for k in range(5):
        off = s * 640 + k * 128
        pltpu.async_copy(acc_sh.at[pl.ds(off, 128)], out_hbm.at[c, pl.ds(off, 128)], sout)
    for k in range(5):
        off = s * 640 + k * 128
        pltpu.make_async_copy(acc_sh.at[pl.ds(off, 128)], out_hbm.at[c, pl.ds(off, 128)], sout).wait()


R = 2000
G = N // R


def _tc1_body(hist_ref, x_ref, w_ref, y_ref):
    dinv = lax.rsqrt(hist_ref[0] + hist_ref[1] + 1.0)
    y_ref[...] = dinv * jnp.dot(x_ref[...], w_ref[...], preferred_element_type=jnp.float32)


def _tc2_body(hist_ref, agg_ref, y1_ref, b1_ref, w2_ref, y2_ref):
    dinv = lax.rsqrt(hist_ref[0] + hist_ref[1] + 1.0)
    pre = agg_ref[0] + agg_ref[1] + y1_ref[...]
    h = jnp.maximum(dinv * pre + b1_ref[...], 0.0)
    y2_ref[...] = dinv * jnp.dot(h, w2_ref[...], preferred_element_type=jnp.float32)


def _tc3_body(hist_ref, agg_ref, y2_ref, b2_ref, out_ref):
    dinv = lax.rsqrt(hist_ref[0] + hist_ref[1] + 1.0)
    out_ref[...] = dinv * (agg_ref[0] + agg_ref[1] + y2_ref[...]) + b2_ref[...]


_hist_spec = pl.BlockSpec((2, R, 1), lambda i: (0, i, 0))
_row_spec = pl.BlockSpec((R, D), lambda i: (i, 0))
_agg_spec = pl.BlockSpec((2, R, D), lambda i: (0, i, 0))
_w_spec = pl.BlockSpec((D, D), lambda i: (0, 0))
_b_spec = pl.BlockSpec((1, D), lambda i: (0, 0))
_row_out = jax.ShapeDtypeStruct((N, D), jnp.float32)

_tc1 = pl.pallas_call(
    _tc1_body, grid=(G,),
    in_specs=[_hist_spec, _row_spec, _w_spec],
    out_specs=_row_spec, out_shape=_row_out,
)
_tc2 = pl.pallas_call(
    _tc2_body, grid=(G,),
    in_specs=[_hist_spec, _agg_spec, _row_spec, _b_spec, _w_spec],
    out_specs=_row_spec, out_shape=_row_out,
)
_tc3 = pl.pallas_call(
    _tc3_body, grid=(G,),
    in_specs=[_hist_spec, _agg_spec, _row_spec, _b_spec],
    out_specs=_row_spec, out_shape=_row_out,
)


@jax.jit
def _impl(x, edge_index, W1, b1, W2, b2):
    src3 = edge_index[0].reshape(NW * NP_, CH, K)
    dst3 = edge_index[1].reshape(NW * NP_, CH, K)
    histp = _hist(dst3)
    hist = histp.reshape(2, NPAD, 1)[:, :N]
    y1 = _tc1(hist, x, W1)
    agg1 = _agg(y1, src3, dst3)
    y2 = _tc2(hist, agg1, y1, b1.reshape(1, D), W2)
    agg2 = _agg(y2, src3, dst3)
    return _tc3(hist, agg2, y2, b2.reshape(1, D))


def kernel(x, edge_index, W1, b1, W2, b2):
    return _impl(x, edge_index, W1, b1, W2, b2)
